# Initial kernel scaffold; baseline (speedup 1.0000x reference)
import jax
import jax.numpy as jnp
from jax.experimental import pallas as pl


def kernel(x, weight, bias, gamma1, beta1, gamma2, beta2):
    raise NotImplementedError("write your pallas kernel here")


# speedup vs baseline: 3.7617x; 3.7617x over previous
import functools

import jax
import jax.numpy as jnp
from jax.experimental import pallas as pl
from jax.experimental.pallas import tpu as pltpu

EPS = 1e-5
STRIDE = 2


def _fused_kernel(x_ref, w_ref, b_ref, g1_ref, b1_ref, g2_ref, b2_ref, o_ref,
                  *, inv_area):
    s = jnp.sum(x_ref[0], axis=1)
    w = jnp.sum(w_ref[...], axis=0)
    m = jnp.dot(s[None, :], w, preferred_element_type=jnp.float32)
    m = m * inv_area + b_ref[...]

    mu = jnp.mean(m, axis=-1, keepdims=True)
    var = jnp.mean(jnp.square(m - mu), axis=-1, keepdims=True)
    h = (m - mu) * jax.lax.rsqrt(var + EPS) * g1_ref[...] + b1_ref[...]
    h = jnp.clip(h, -1.0, 1.0)

    mu2 = jnp.mean(h, axis=-1, keepdims=True)
    var2 = jnp.mean(jnp.square(h - mu2), axis=-1, keepdims=True)
    o_ref[0] = (h - mu2) * jax.lax.rsqrt(var2 + EPS) * g2_ref[...] + b2_ref[...]


def kernel(x, weight, bias, gamma1, beta1, gamma2, beta2):
    N, Cin, H, W = x.shape
    Cout, K = weight.shape[1], weight.shape[2]
    Hout = (H - 1) * STRIDE + K
    Wout = (W - 1) * STRIDE + K
    inv_area = 1.0 / float(Hout * Wout)

    xr = x.reshape(N, Cin, H * W)
    wr = jnp.transpose(weight, (2, 3, 0, 1)).reshape(K * K, Cin, Cout)
    vec = lambda v: v.reshape(1, Cout)

    return pl.pallas_call(
        functools.partial(_fused_kernel, inv_area=inv_area),
        grid=(N,),
        in_specs=[
            pl.BlockSpec((1, Cin, H * W), lambda n: (n, 0, 0)),
            pl.BlockSpec((K * K, Cin, Cout), lambda n: (0, 0, 0)),
            pl.BlockSpec((1, Cout), lambda n: (0, 0)),
            pl.BlockSpec((1, Cout), lambda n: (0, 0)),
            pl.BlockSpec((1, Cout), lambda n: (0, 0)),
            pl.BlockSpec((1, Cout), lambda n: (0, 0)),
            pl.BlockSpec((1, Cout), lambda n: (0, 0)),
        ],
        out_specs=pl.BlockSpec((1, 1, Cout), lambda n: (n, 0, 0)),
        out_shape=jax.ShapeDtypeStruct((N, 1, Cout), jnp.float32),
        compiler_params=pltpu.CompilerParams(
            dimension_semantics=("parallel",)),
    )(xr, wr, vec(bias), vec(gamma1), vec(beta1), vec(gamma2),
      vec(beta2)).reshape(N, Cout)


# speedup vs baseline: 7.9810x; 2.1216x over previous
import functools

import jax
import jax.numpy as jnp
from jax.experimental import pallas as pl
from jax.experimental.pallas import tpu as pltpu

EPS = 1e-5
STRIDE = 2


def _fused_kernel(x_ref, w_ref, b_ref, g1_ref, b1_ref, g2_ref, b2_ref, o_ref,
                  *, inv_area):
    s = jnp.sum(x_ref[0], axis=(1, 2))
    w = jnp.sum(w_ref[...], axis=0)
    m = jnp.dot(s[None, :], w, preferred_element_type=jnp.float32)
    m = m * inv_area + b_ref[...]

    mu = jnp.mean(m, axis=-1, keepdims=True)
    var = jnp.mean(jnp.square(m - mu), axis=-1, keepdims=True)
    h = (m - mu) * jax.lax.rsqrt(var + EPS) * g1_ref[...] + b1_ref[...]
    h = jnp.clip(h, -1.0, 1.0)

    mu2 = jnp.mean(h, axis=-1, keepdims=True)
    var2 = jnp.mean(jnp.square(h - mu2), axis=-1, keepdims=True)
    o_ref[0] = (h - mu2) * jax.lax.rsqrt(var2 + EPS) * g2_ref[...] + b2_ref[...]


def kernel(x, weight, bias, gamma1, beta1, gamma2, beta2):
    N, Cin, H, W = x.shape
    Cout, K = weight.shape[1], weight.shape[2]
    Hout = (H - 1) * STRIDE + K
    Wout = (W - 1) * STRIDE + K
    inv_area = 1.0 / float(Hout * Wout)

    wr = jnp.transpose(weight, (2, 3, 0, 1)).reshape(K * K, Cin, Cout)
    vec = lambda v: v.reshape(1, Cout)

    return pl.pallas_call(
        functools.partial(_fused_kernel, inv_area=inv_area),
        grid=(N,),
        in_specs=[
            pl.BlockSpec((1, Cin, H, W), lambda n: (n, 0, 0, 0)),
            pl.BlockSpec((K * K, Cin, Cout), lambda n: (0, 0, 0)),
            pl.BlockSpec((1, Cout), lambda n: (0, 0)),
            pl.BlockSpec((1, Cout), lambda n: (0, 0)),
            pl.BlockSpec((1, Cout), lambda n: (0, 0)),
            pl.BlockSpec((1, Cout), lambda n: (0, 0)),
            pl.BlockSpec((1, Cout), lambda n: (0, 0)),
        ],
        out_specs=pl.BlockSpec((1, 1, Cout), lambda n: (n, 0, 0)),
        out_shape=jax.ShapeDtypeStruct((N, 1, Cout), jnp.float32),
        compiler_params=pltpu.CompilerParams(
            dimension_semantics=("parallel",)),
    )(x, wr, vec(bias), vec(gamma1), vec(beta1), vec(gamma2),
      vec(beta2)).reshape(N, Cout)


# speedup vs baseline: 12.7978x; 1.6035x over previous
import functools

import jax
import jax.numpy as jnp
from jax.experimental import pallas as pl
from jax.experimental.pallas import tpu as pltpu

EPS = 1e-5
STRIDE = 2
BN = 8


def _fused_kernel(x_ref, w_ref, b_ref, g1_ref, b1_ref, g2_ref, b2_ref, o_ref,
                  *, inv_area):
    s = jnp.sum(x_ref[...], axis=(2, 3))
    w = jnp.sum(w_ref[...], axis=0)
    m = jnp.dot(s, w, preferred_element_type=jnp.float32)
    m = m * inv_area + b_ref[...]

    mu = jnp.mean(m, axis=-1, keepdims=True)
    var = jnp.mean(jnp.square(m - mu), axis=-1, keepdims=True)
    h = (m - mu) * jax.lax.rsqrt(var + EPS) * g1_ref[...] + b1_ref[...]
    h = jnp.clip(h, -1.0, 1.0)

    mu2 = jnp.mean(h, axis=-1, keepdims=True)
    var2 = jnp.mean(jnp.square(h - mu2), axis=-1, keepdims=True)
    o_ref[...] = (h - mu2) * jax.lax.rsqrt(var2 + EPS) * g2_ref[...] + b2_ref[...]


def kernel(x, weight, bias, gamma1, beta1, gamma2, beta2):
    N, Cin, H, W = x.shape
    Cout, K = weight.shape[1], weight.shape[2]
    Hout = (H - 1) * STRIDE + K
    Wout = (W - 1) * STRIDE + K
    inv_area = 1.0 / float(Hout * Wout)

    wr = jnp.transpose(weight, (2, 3, 0, 1)).reshape(K * K, Cin, Cout)
    vec = lambda v: v.reshape(1, Cout)

    return pl.pallas_call(
        functools.partial(_fused_kernel, inv_area=inv_area),
        grid=(N // BN,),
        in_specs=[
            pl.BlockSpec((BN, Cin, H, W), lambda n: (n, 0, 0, 0)),
            pl.BlockSpec((K * K, Cin, Cout), lambda n: (0, 0, 0)),
            pl.BlockSpec((1, Cout), lambda n: (0, 0)),
            pl.BlockSpec((1, Cout), lambda n: (0, 0)),
            pl.BlockSpec((1, Cout), lambda n: (0, 0)),
            pl.BlockSpec((1, Cout), lambda n: (0, 0)),
            pl.BlockSpec((1, Cout), lambda n: (0, 0)),
        ],
        out_specs=pl.BlockSpec((BN, Cout), lambda n: (n, 0)),
        out_shape=jax.ShapeDtypeStruct((N, Cout), jnp.float32),
        compiler_params=pltpu.CompilerParams(
            dimension_semantics=("parallel",)),
    )(x, wr, vec(bias), vec(gamma1), vec(beta1), vec(gamma2), vec(beta2))
